# Initial kernel scaffold; baseline (speedup 1.0000x reference)
import jax
import jax.numpy as jnp
from jax.experimental import pallas as pl


def kernel(x, edge_index, W1, b1, W2, b2):
    raise NotImplementedError("write your pallas kernel here")


# speedup vs baseline: 70.8624x; 70.8624x over previous
import functools

import jax
import jax.numpy as jnp
from jax import lax
from jax.experimental import pallas as pl
from jax.experimental.pallas import tpu as pltpu
from jax.experimental.pallas import tpu_sc as plsc

NUM_CORES = 2
NUM_SUBCORES = 16
NUM_WORKERS = NUM_CORES * NUM_SUBCORES
CHUNK = 128
TAIL = 16

_NBUF = 6
_LAG = 2


def _mm_body(x_ref, w_ref, o_ref):
    o_ref[...] = jnp.dot(x_ref[...], w_ref[...],
                         preferred_element_type=jnp.float32)


def _matmul(x, w, block_rows):
    n, d = x.shape
    h = w.shape[1]
    return pl.pallas_call(
        _mm_body,
        grid=(n // block_rows,),
        in_specs=[
            pl.BlockSpec((block_rows, d), lambda i: (i, 0)),
            pl.BlockSpec((d, h), lambda i: (0, 0)),
        ],
        out_specs=pl.BlockSpec((block_rows, h), lambda i: (i, 0)),
        out_shape=jax.ShapeDtypeStruct((n, h), jnp.float32),
    )(x, w)


def _scale_body(xw_ref, hist_ref, y_ref, dis_ref):
    deg = jnp.sum(hist_ref[...], axis=0) + 1.0
    dis = lax.rsqrt(deg)[:, None]
    dis_ref[...] = jnp.broadcast_to(dis, dis_ref.shape)
    y_ref[...] = xw_ref[...] * dis


def _scale(xw, hist):
    n, h = xw.shape
    return pl.pallas_call(
        _scale_body,
        out_shape=[
            jax.ShapeDtypeStruct((n, h), jnp.float32),
            jax.ShapeDtypeStruct((n, h), jnp.float32),
        ],
    )(xw, hist)


def _final_body(p0_ref, p1_ref, y_ref, dis_ref, b1_ref, w2_ref, b2_ref, o_ref):
    h = dis_ref[...] * (p0_ref[0] + p1_ref[0] + y_ref[...]) + b1_ref[...]
    h = jnp.maximum(h, 0.0)
    o_ref[...] = jnp.dot(h, w2_ref[...],
                         preferred_element_type=jnp.float32) + b2_ref[...]


def _final(parts, y, dis, b1, w2, b2, block_rows):
    n, h = y.shape
    c = w2.shape[1]
    spec = pl.BlockSpec((block_rows, h), lambda i: (i, 0))
    return pl.pallas_call(
        _final_body,
        grid=(n // block_rows,),
        in_specs=[
            pl.BlockSpec((1, block_rows, h), lambda i: (0, i, 0)),
            pl.BlockSpec((1, block_rows, h), lambda i: (1, i, 0)),
            spec, spec,
            pl.BlockSpec((1, h), lambda i: (0, 0)),
            pl.BlockSpec((h, c), lambda i: (0, 0)),
            pl.BlockSpec((1, c), lambda i: (0, 0)),
        ],
        out_specs=pl.BlockSpec((block_rows, c), lambda i: (i, 0)),
        out_shape=jax.ShapeDtypeStruct((n, c), jnp.float32),
    )(parts, parts, y, dis, b1, w2, b2)


def _sc_mesh():
    return plsc.VectorSubcoreMesh(
        core_axis_name="c", subcore_axis_name="s",
        num_cores=NUM_CORES, num_subcores=NUM_SUBCORES)


_SC_PARAMS = pltpu.CompilerParams(use_tc_tiling_on_sc=False)
_SC_PARAMS_NOLAYOUT = pltpu.CompilerParams(
    use_tc_tiling_on_sc=False, needs_layout_passes=False)


def _edge_split(e):
    chunks = e // CHUNK
    mc = chunks // NUM_WORKERS
    lb = chunks - mc * NUM_WORKERS
    return chunks, mc, lb


def _sc_hist(edge3, n, e):
    chunks, mc, lb = _edge_split(e)

    @functools.partial(
        pl.kernel,
        out_type=jax.ShapeDtypeStruct((NUM_WORKERS, n), jnp.float32),
        mesh=_sc_mesh(),
        compiler_params=_SC_PARAMS_NOLAYOUT,
        scratch_types=[
            pltpu.VMEM((mc, CHUNK), jnp.int32),
            pltpu.VMEM((CHUNK,), jnp.int32),
            pltpu.VMEM((n,), jnp.float32),
        ],
    )
    def k(edge3_hbm, out_hbm, idx_v, idxt_v, cnt_v):
        cid = lax.axis_index("c")
        sid = lax.axis_index("s")
        w = cid * NUM_SUBCORES + sid
        zeros16 = jnp.zeros((16,), jnp.float32)

        @pl.loop(0, n // 16)
        def _(i):
            cnt_v[pl.ds(i * 16, 16)] = zeros16

        pltpu.sync_copy(edge3_hbm.at[1, pl.ds(w * mc, mc)], idx_v)
        ones16 = jnp.ones((16,), jnp.float32)

        @pl.loop(0, mc)
        def _(r):
            @pl.loop(0, CHUNK // 16)
            def _(c):
                idx16 = idx_v[r, pl.ds(c * 16, 16)]
                plsc.addupdate_scatter(cnt_v, [idx16], ones16)

        if lb:
            @pl.when(w < lb)
            def _():
                pltpu.sync_copy(edge3_hbm.at[1, mc * NUM_WORKERS + w], idxt_v)

                @pl.loop(0, CHUNK // 16)
                def _(c):
                    idx16 = idxt_v[pl.ds(c * 16, 16)]
                    plsc.addupdate_scatter(cnt_v, [idx16], ones16)

        pltpu.sync_copy(cnt_v, out_hbm.at[w])

    return k(edge3)


def _sc_scatter(edge3, y, zeros_init, n, e, h):
    chunks, mc, lb = _edge_split(e)
    rows_per_sub = n // NUM_SUBCORES
    n_groups = mc // _NBUF
    rem = mc - n_groups * _NBUF

    @functools.partial(
        pl.kernel,
        out_type=jax.ShapeDtypeStruct((NUM_CORES, n, h), jnp.float32),
        mesh=_sc_mesh(),
        compiler_params=_SC_PARAMS,
        scratch_types=[
            pltpu.VMEM((mc, CHUNK), jnp.int32),
            pltpu.VMEM((mc, CHUNK), jnp.int32),
            pltpu.VMEM((CHUNK,), jnp.int32),
            pltpu.VMEM((CHUNK,), jnp.int32),
            pltpu.VMEM((_NBUF, CHUNK, h), jnp.float32),
            pltpu.VMEM((CHUNK, h), jnp.float32),
            pltpu.VMEM_SHARED((n, h), jnp.float32),
            pltpu.SemaphoreType.DMA,
            pltpu.SemaphoreType.DMA,
        ],
    )
    def k(edge3_hbm, y_hbm, zeros_hbm, out_hbm,
          idxs_v, idxd_v, idxts_v, idxtd_v, rows_v, rowst_v, acc_sh,
          sem_g, sem_s):
        cid = lax.axis_index("c")
        sid = lax.axis_index("s")
        w = cid * NUM_SUBCORES + sid
        row0 = sid * rows_per_sub
        pltpu.sync_copy(zeros_hbm.at[pl.ds(row0, rows_per_sub)],
                        acc_sh.at[pl.ds(row0, rows_per_sub)])
        pltpu.sync_copy(edge3_hbm.at[0, pl.ds(w * mc, mc)], idxs_v)
        pltpu.sync_copy(edge3_hbm.at[1, pl.ds(w * mc, mc)], idxd_v)
        plsc.subcore_barrier()

        for b in range(_NBUF - _LAG):
            pltpu.async_copy(y_hbm.at[idxs_v.at[b]], rows_v.at[b], sem_g)

        @pl.loop(0, n_groups)
        def _(o):
            for b in range(_NBUF):
                i = o * _NBUF + b
                pltpu.make_async_copy(
                    y_hbm.at[idxs_v.at[i]], rows_v.at[b], sem_g).wait()
                pltpu.async_copy(rows_v.at[b], acc_sh.at[idxd_v.at[i]],
                                 sem_s, add=True)
                refill = i + _NBUF - _LAG
                bb = (b - _LAG) % _NBUF

                @pl.when(refill < mc)
                def _():
                    @pl.when(i >= _LAG)
                    def _():
                        pltpu.make_async_copy(
                            rows_v.at[bb], acc_sh.at[idxd_v.at[i]],
                            sem_s).wait()
                    pltpu.async_copy(y_hbm.at[idxs_v.at[refill]],
                                     rows_v.at[bb], sem_g)

        for b in range(min(_NBUF, mc)):
            pltpu.make_async_copy(rows_v.at[b], acc_sh.at[idxd_v.at[b]],
                                  sem_s).wait()

        for r in range(rem):
            i = n_groups * _NBUF + r
            pltpu.async_copy(y_hbm.at[idxs_v.at[i]], rows_v.at[0],
                             sem_g).wait()
            pltpu.sync_copy(rows_v.at[0], acc_sh.at[idxd_v.at[i]], add=True)

        if lb:
            @pl.when(w < lb)
            def _():
                blk = mc * NUM_WORKERS + w
                pltpu.sync_copy(edge3_hbm.at[0, blk], idxts_v)
                pltpu.sync_copy(edge3_hbm.at[1, blk], idxtd_v)
                pltpu.async_copy(y_hbm.at[idxts_v], rowst_v, sem_g).wait()
                pltpu.sync_copy(rowst_v, acc_sh.at[idxtd_v], add=True)

        plsc.subcore_barrier()
        pltpu.sync_copy(acc_sh.at[pl.ds(row0, rows_per_sub)],
                        out_hbm.at[cid, pl.ds(row0, rows_per_sub)])

    return k(edge3, y, zeros_init)


def kernel(x, edge_index, W1, b1, W2, b2):
    n, d = x.shape
    h = W1.shape[1]
    c = W2.shape[1]
    e = edge_index.shape[1]

    edge3 = edge_index.astype(jnp.int32).reshape(2, e // CHUNK, CHUNK)
    zeros_init = jnp.zeros((n, h), jnp.float32)

    block_rows = 2000 if n % 2000 == 0 else 8 * (n // 8)

    xw = _matmul(x, W1, block_rows)
    hist = _sc_hist(edge3, n, e)
    y, dis = _scale(xw, hist)
    parts = _sc_scatter(edge3, y, zeros_init, n, e, h)
    return _final(parts, y, dis,
                  b1.reshape(1, h), W2, b2.reshape(1, c), block_rows)


# speedup vs baseline: 77.1007x; 1.0880x over previous
import functools

import jax
import jax.numpy as jnp
from jax import lax
from jax.experimental import pallas as pl
from jax.experimental.pallas import tpu as pltpu
from jax.experimental.pallas import tpu_sc as plsc

NUM_CORES = 2
NUM_SUBCORES = 16
NUM_WORKERS = NUM_CORES * NUM_SUBCORES
CHUNK = 128
TAIL = 16

_NBUF = 6
_LAG = 2


def _mm_body(x_ref, w_ref, o_ref):
    o_ref[...] = jnp.dot(x_ref[...], w_ref[...],
                         preferred_element_type=jnp.float32)


def _matmul(x, w, block_rows):
    n, d = x.shape
    h = w.shape[1]
    return pl.pallas_call(
        _mm_body,
        grid=(n // block_rows,),
        in_specs=[
            pl.BlockSpec((block_rows, d), lambda i: (i, 0)),
            pl.BlockSpec((d, h), lambda i: (0, 0)),
        ],
        out_specs=pl.BlockSpec((block_rows, h), lambda i: (i, 0)),
        out_shape=jax.ShapeDtypeStruct((n, h), jnp.float32),
    )(x, w)


def _scale_body(xw_ref, hist_ref, y_ref, dis_ref):
    n = xw_ref.shape[0]
    deg = jnp.sum(hist_ref[...], axis=0)[:n] + 1.0
    dis = lax.rsqrt(deg)[:, None]
    dis_ref[...] = jnp.broadcast_to(dis, dis_ref.shape)
    y_ref[...] = xw_ref[...] * dis


def _scale(xw, hist):
    n, h = xw.shape
    return pl.pallas_call(
        _scale_body,
        out_shape=[
            jax.ShapeDtypeStruct((n, h), jnp.float32),
            jax.ShapeDtypeStruct((n, h), jnp.float32),
        ],
    )(xw, hist)


def _final_body(p_ref, y_ref, dis_ref, b1_ref, w2_ref, b2_ref, o_ref):
    h = dis_ref[...] * (p_ref[0] + p_ref[1] + y_ref[...]) + b1_ref[...]
    h = jnp.maximum(h, 0.0)
    o_ref[...] = jnp.dot(h, w2_ref[...],
                         preferred_element_type=jnp.float32) + b2_ref[...]


def _final(parts2, y2, dis2, b1p, w2p, b2p):
    nr, _ = y2.shape
    cp = w2p.shape[1]
    return pl.pallas_call(
        _final_body,
        out_shape=jax.ShapeDtypeStruct((nr, cp), jnp.float32),
    )(parts2, y2, dis2, b1p, w2p, b2p)


def _sc_mesh():
    return plsc.VectorSubcoreMesh(
        core_axis_name="c", subcore_axis_name="s",
        num_cores=NUM_CORES, num_subcores=NUM_SUBCORES)


_SC_PARAMS = pltpu.CompilerParams(use_tc_tiling_on_sc=False)
_SC_PARAMS_NOLAYOUT = pltpu.CompilerParams(
    use_tc_tiling_on_sc=False, needs_layout_passes=False)


def _edge_split(e):
    chunks = e // CHUNK
    mc = chunks // NUM_WORKERS
    lb = chunks - mc * NUM_WORKERS
    return chunks, mc, lb


def _sc_hist(edge3, n, e):
    chunks, mc, lb = _edge_split(e)
    nh = -(-n // 128) * 128

    @functools.partial(
        pl.kernel,
        out_type=jax.ShapeDtypeStruct((NUM_WORKERS, nh), jnp.float32),
        mesh=_sc_mesh(),
        compiler_params=_SC_PARAMS_NOLAYOUT,
        scratch_types=[
            pltpu.VMEM((mc, CHUNK), jnp.int32),
            pltpu.VMEM((CHUNK,), jnp.int32),
            pltpu.VMEM((nh,), jnp.float32),
        ],
    )
    def k(edge3_hbm, out_hbm, idx_v, idxt_v, cnt_v):
        cid = lax.axis_index("c")
        sid = lax.axis_index("s")
        w = cid * NUM_SUBCORES + sid
        zeros16 = jnp.zeros((16,), jnp.float32)

        @pl.loop(0, nh // 16)
        def _(i):
            cnt_v[pl.ds(i * 16, 16)] = zeros16

        pltpu.sync_copy(edge3_hbm.at[1, pl.ds(w * mc, mc)], idx_v)
        ones16 = jnp.ones((16,), jnp.float32)

        @pl.loop(0, mc)
        def _(r):
            @pl.loop(0, CHUNK // 16)
            def _(c):
                idx16 = idx_v[r, pl.ds(c * 16, 16)]
                plsc.addupdate_scatter(cnt_v, [idx16], ones16)

        if lb:
            @pl.when(w < lb)
            def _():
                pltpu.sync_copy(edge3_hbm.at[1, mc * NUM_WORKERS + w], idxt_v)

                @pl.loop(0, CHUNK // 16)
                def _(c):
                    idx16 = idxt_v[pl.ds(c * 16, 16)]
                    plsc.addupdate_scatter(cnt_v, [idx16], ones16)

        pltpu.sync_copy(cnt_v, out_hbm.at[w])

    return k(edge3)


def _sc_scatter(edge3, y, zeros_init, n, e, h):
    chunks, mc, lb = _edge_split(e)
    rows_per_sub = n // NUM_SUBCORES
    n_groups = mc // _NBUF
    rem = mc - n_groups * _NBUF

    @functools.partial(
        pl.kernel,
        out_type=jax.ShapeDtypeStruct((NUM_CORES, n, h), jnp.float32),
        mesh=_sc_mesh(),
        compiler_params=_SC_PARAMS,
        scratch_types=[
            pltpu.VMEM((mc, CHUNK), jnp.int32),
            pltpu.VMEM((mc, CHUNK), jnp.int32),
            pltpu.VMEM((CHUNK,), jnp.int32),
            pltpu.VMEM((CHUNK,), jnp.int32),
            pltpu.VMEM((_NBUF, CHUNK, h), jnp.float32),
            pltpu.VMEM((CHUNK, h), jnp.float32),
            pltpu.VMEM_SHARED((n, h), jnp.float32),
            pltpu.SemaphoreType.DMA,
            pltpu.SemaphoreType.DMA,
        ],
    )
    def k(edge3_hbm, y_hbm, zeros_hbm, out_hbm,
          idxs_v, idxd_v, idxts_v, idxtd_v, rows_v, rowst_v, acc_sh,
          sem_g, sem_s):
        cid = lax.axis_index("c")
        sid = lax.axis_index("s")
        w = cid * NUM_SUBCORES + sid
        row0 = sid * rows_per_sub
        pltpu.sync_copy(zeros_hbm.at[pl.ds(row0, rows_per_sub)],
                        acc_sh.at[pl.ds(row0, rows_per_sub)])
        pltpu.sync_copy(edge3_hbm.at[0, pl.ds(w * mc, mc)], idxs_v)
        pltpu.sync_copy(edge3_hbm.at[1, pl.ds(w * mc, mc)], idxd_v)
        plsc.subcore_barrier()

        for b in range(_NBUF - _LAG):
            pltpu.async_copy(y_hbm.at[idxs_v.at[b]], rows_v.at[b], sem_g)

        @pl.loop(0, n_groups)
        def _(o):
            for b in range(_NBUF):
                i = o * _NBUF + b
                pltpu.make_async_copy(
                    y_hbm.at[idxs_v.at[i]], rows_v.at[b], sem_g).wait()
                pltpu.async_copy(rows_v.at[b], acc_sh.at[idxd_v.at[i]],
                                 sem_s, add=True)
                refill = i + _NBUF - _LAG
                bb = (b - _LAG) % _NBUF

                @pl.when(refill < mc)
                def _():
                    @pl.when(i >= _LAG)
                    def _():
                        pltpu.make_async_copy(
                            rows_v.at[bb], acc_sh.at[idxd_v.at[i]],
                            sem_s).wait()
                    pltpu.async_copy(y_hbm.at[idxs_v.at[refill]],
                                     rows_v.at[bb], sem_g)

        for b in range(min(_NBUF, mc)):
            pltpu.make_async_copy(rows_v.at[b], acc_sh.at[idxd_v.at[b]],
                                  sem_s).wait()

        for r in range(rem):
            i = n_groups * _NBUF + r
            pltpu.async_copy(y_hbm.at[idxs_v.at[i]], rows_v.at[0],
                             sem_g).wait()
            pltpu.sync_copy(rows_v.at[0], acc_sh.at[idxd_v.at[i]], add=True)

        if lb:
            @pl.when(w < lb)
            def _():
                blk = mc * NUM_WORKERS + w
                pltpu.sync_copy(edge3_hbm.at[0, blk], idxts_v)
                pltpu.sync_copy(edge3_hbm.at[1, blk], idxtd_v)
                pltpu.async_copy(y_hbm.at[idxts_v], rowst_v, sem_g).wait()
                pltpu.sync_copy(rowst_v, acc_sh.at[idxtd_v], add=True)

        plsc.subcore_barrier()
        pltpu.sync_copy(acc_sh.at[pl.ds(row0, rows_per_sub)],
                        out_hbm.at[cid, pl.ds(row0, rows_per_sub)])

    return k(edge3, y, zeros_init)


def kernel(x, edge_index, W1, b1, W2, b2):
    n, d = x.shape
    h = W1.shape[1]
    c = W2.shape[1]
    e = edge_index.shape[1]

    edge3 = edge_index.astype(jnp.int32).reshape(2, e // CHUNK, CHUNK)
    zeros_init = jnp.zeros((n, h), jnp.float32)

    block_rows = 2000 if n % 2000 == 0 else 8 * (n // 8)
    pack = 128 // h

    xw = _matmul(x, W1, block_rows)
    hist = _sc_hist(edge3, n, e)
    y, dis = _scale(xw, hist)
    parts = _sc_scatter(edge3, y, zeros_init, n, e, h)

    parts2 = parts.reshape(NUM_CORES, n // pack, 128)
    y2 = y.reshape(n // pack, 128)
    dis2 = dis.reshape(n // pack, 128)
    w2p = jnp.kron(jnp.eye(pack, dtype=jnp.float32), W2)
    b1p = jnp.tile(b1, pack).reshape(1, 128)
    b2p = jnp.tile(b2, pack).reshape(1, pack * c)
    out2 = _final(parts2, y2, dis2, b1p, w2p, b2p)
    return out2.reshape(n, c)


# speedup vs baseline: 88.6731x; 1.1501x over previous
import functools

import jax
import jax.numpy as jnp
from jax import lax
from jax.experimental import pallas as pl
from jax.experimental.pallas import tpu as pltpu
from jax.experimental.pallas import tpu_sc as plsc

NUM_CORES = 2
NUM_SUBCORES = 16
NUM_WORKERS = NUM_CORES * NUM_SUBCORES
CHUNK = 128
TAIL = 16

_NBUF = 13
_LAG = 3


def _mm_body(x_ref, w_ref, o_ref):
    o_ref[...] = jnp.dot(x_ref[...], w_ref[...],
                         preferred_element_type=jnp.float32)


def _matmul(x, w, block_rows):
    n, d = x.shape
    h = w.shape[1]
    return pl.pallas_call(
        _mm_body,
        grid=(n // block_rows,),
        in_specs=[
            pl.BlockSpec((block_rows, d), lambda i: (i, 0)),
            pl.BlockSpec((d, h), lambda i: (0, 0)),
        ],
        out_specs=pl.BlockSpec((block_rows, h), lambda i: (i, 0)),
        out_shape=jax.ShapeDtypeStruct((n, h), jnp.float32),
    )(x, w)


def _scale_body(xw_ref, hist_ref, y_ref, dis_ref):
    n = xw_ref.shape[0]
    deg = jnp.sum(hist_ref[...], axis=0)[:n] + 1.0
    dis = lax.rsqrt(deg)[:, None]
    dis_ref[...] = jnp.broadcast_to(dis, dis_ref.shape)
    y_ref[...] = xw_ref[...] * dis


def _scale(xw, hist):
    n, h = xw.shape
    return pl.pallas_call(
        _scale_body,
        out_shape=[
            jax.ShapeDtypeStruct((n, h), jnp.float32),
            jax.ShapeDtypeStruct((n, h), jnp.float32),
        ],
    )(xw, hist)


def _final_body(p_ref, y_ref, dis_ref, b1_ref, w2_ref, b2_ref, o_ref):
    h = dis_ref[...] * (p_ref[0] + p_ref[1] + y_ref[...]) + b1_ref[...]
    h = jnp.maximum(h, 0.0)
    o_ref[...] = jnp.dot(h, w2_ref[...],
                         preferred_element_type=jnp.float32) + b2_ref[...]


def _final(parts2, y2, dis2, b1p, w2p, b2p):
    nr, _ = y2.shape
    cp = w2p.shape[1]
    return pl.pallas_call(
        _final_body,
        out_shape=jax.ShapeDtypeStruct((nr, cp), jnp.float32),
    )(parts2, y2, dis2, b1p, w2p, b2p)


def _sc_mesh():
    return plsc.VectorSubcoreMesh(
        core_axis_name="c", subcore_axis_name="s",
        num_cores=NUM_CORES, num_subcores=NUM_SUBCORES)


_SC_PARAMS = pltpu.CompilerParams(use_tc_tiling_on_sc=False)
_SC_PARAMS_NOLAYOUT = pltpu.CompilerParams(
    use_tc_tiling_on_sc=False, needs_layout_passes=False)


def _edge_split(e):
    chunks = e // CHUNK
    mc = chunks // NUM_WORKERS
    lb = chunks - mc * NUM_WORKERS
    return chunks, mc, lb


def _sc_hist(edge3, n, e):
    chunks, mc, lb = _edge_split(e)
    nh = -(-n // 128) * 128

    @functools.partial(
        pl.kernel,
        out_type=jax.ShapeDtypeStruct((NUM_WORKERS, nh), jnp.float32),
        mesh=_sc_mesh(),
        compiler_params=_SC_PARAMS_NOLAYOUT,
        scratch_types=[
            pltpu.VMEM((mc, CHUNK), jnp.int32),
            pltpu.VMEM((CHUNK,), jnp.int32),
            pltpu.VMEM((nh,), jnp.float32),
        ],
    )
    def k(edge3_hbm, out_hbm, idx_v, idxt_v, cnt_v):
        cid = lax.axis_index("c")
        sid = lax.axis_index("s")
        w = cid * NUM_SUBCORES + sid
        zeros16 = jnp.zeros((16,), jnp.float32)

        @pl.loop(0, nh // 16)
        def _(i):
            cnt_v[pl.ds(i * 16, 16)] = zeros16

        pltpu.sync_copy(edge3_hbm.at[1, pl.ds(w * mc, mc)], idx_v)
        ones16 = jnp.ones((16,), jnp.float32)

        @pl.loop(0, mc)
        def _(r):
            @pl.loop(0, CHUNK // 16)
            def _(c):
                idx16 = idx_v[r, pl.ds(c * 16, 16)]
                plsc.addupdate_scatter(cnt_v, [idx16], ones16)

        if lb:
            @pl.when(w < lb)
            def _():
                pltpu.sync_copy(edge3_hbm.at[1, mc * NUM_WORKERS + w], idxt_v)

                @pl.loop(0, CHUNK // 16)
                def _(c):
                    idx16 = idxt_v[pl.ds(c * 16, 16)]
                    plsc.addupdate_scatter(cnt_v, [idx16], ones16)

        pltpu.sync_copy(cnt_v, out_hbm.at[w])

    return k(edge3)


def _sc_scatter(edge3, y, zeros_init, n, e, h):
    chunks, mc, lb = _edge_split(e)
    rows_per_sub = n // NUM_SUBCORES
    n_groups = mc // _NBUF
    rem = mc - n_groups * _NBUF

    @functools.partial(
        pl.kernel,
        out_type=jax.ShapeDtypeStruct((NUM_CORES, n, h), jnp.float32),
        mesh=_sc_mesh(),
        compiler_params=_SC_PARAMS,
        scratch_types=[
            pltpu.VMEM((mc, CHUNK), jnp.int32),
            pltpu.VMEM((mc, CHUNK), jnp.int32),
            pltpu.VMEM((CHUNK,), jnp.int32),
            pltpu.VMEM((CHUNK,), jnp.int32),
            pltpu.VMEM((_NBUF, CHUNK, h), jnp.float32),
            pltpu.VMEM((CHUNK, h), jnp.float32),
            pltpu.VMEM_SHARED((n, h), jnp.float32),
            pltpu.SemaphoreType.DMA,
            pltpu.SemaphoreType.DMA,
        ],
    )
    def k(edge3_hbm, y_hbm, zeros_hbm, out_hbm,
          idxs_v, idxd_v, idxts_v, idxtd_v, rows_v, rowst_v, acc_sh,
          sem_g, sem_s):
        cid = lax.axis_index("c")
        sid = lax.axis_index("s")
        w = cid * NUM_SUBCORES + sid
        row0 = sid * rows_per_sub
        pltpu.sync_copy(zeros_hbm.at[pl.ds(row0, rows_per_sub)],
                        acc_sh.at[pl.ds(row0, rows_per_sub)])
        pltpu.sync_copy(edge3_hbm.at[0, pl.ds(w * mc, mc)], idxs_v)
        pltpu.sync_copy(edge3_hbm.at[1, pl.ds(w * mc, mc)], idxd_v)
        plsc.subcore_barrier()

        for b in range(_NBUF - _LAG):
            pltpu.async_copy(y_hbm.at[idxs_v.at[b]], rows_v.at[b], sem_g)

        @pl.loop(0, n_groups)
        def _(o):
            for b in range(_NBUF):
                i = o * _NBUF + b
                pltpu.make_async_copy(
                    y_hbm.at[idxs_v.at[i]], rows_v.at[b], sem_g).wait()
                pltpu.async_copy(rows_v.at[b], acc_sh.at[idxd_v.at[i]],
                                 sem_s, add=True)
                refill = i + _NBUF - _LAG
                bb = (b - _LAG) % _NBUF

                @pl.when(refill < mc)
                def _():
                    @pl.when(i >= _LAG)
                    def _():
                        pltpu.make_async_copy(
                            rows_v.at[bb], acc_sh.at[idxd_v.at[i]],
                            sem_s).wait()
                    pltpu.async_copy(y_hbm.at[idxs_v.at[refill]],
                                     rows_v.at[bb], sem_g)

        for b in range(min(_NBUF, mc)):
            pltpu.make_async_copy(rows_v.at[b], acc_sh.at[idxd_v.at[b]],
                                  sem_s).wait()

        for r in range(rem):
            i = n_groups * _NBUF + r
            pltpu.async_copy(y_hbm.at[idxs_v.at[i]], rows_v.at[0],
                             sem_g).wait()
            pltpu.sync_copy(rows_v.at[0], acc_sh.at[idxd_v.at[i]], add=True)

        if lb:
            @pl.when(w < lb)
            def _():
                blk = mc * NUM_WORKERS + w
                pltpu.sync_copy(edge3_hbm.at[0, blk], idxts_v)
                pltpu.sync_copy(edge3_hbm.at[1, blk], idxtd_v)
                pltpu.async_copy(y_hbm.at[idxts_v], rowst_v, sem_g).wait()
                pltpu.sync_copy(rowst_v, acc_sh.at[idxtd_v], add=True)

        plsc.subcore_barrier()
        pltpu.sync_copy(acc_sh.at[pl.ds(row0, rows_per_sub)],
                        out_hbm.at[cid, pl.ds(row0, rows_per_sub)])

    return k(edge3, y, zeros_init)


def kernel(x, edge_index, W1, b1, W2, b2):
    n, d = x.shape
    h = W1.shape[1]
    c = W2.shape[1]
    e = edge_index.shape[1]

    edge3 = edge_index.astype(jnp.int32).reshape(2, e // CHUNK, CHUNK)
    zeros_init = jnp.zeros((n, h), jnp.float32)

    pack = 128 // h
    block_rows = 2000 if n % 2000 == 0 else 8 * (n // 8)

    xw = _matmul(x, W1, block_rows)
    hist = _sc_hist(edge3, n, e)
    y, dis = _scale(xw, hist)
    parts = _sc_scatter(edge3, y, zeros_init, n, e, h)

    parts2 = parts.reshape(NUM_CORES, n // pack, 128)
    y2 = y.reshape(n // pack, 128)
    dis2 = dis.reshape(n // pack, 128)
    w2p = jnp.kron(jnp.eye(pack, dtype=jnp.float32), W2)
    b1p = jnp.tile(b1, pack).reshape(1, 128)
    b2p = jnp.tile(b2, pack).reshape(1, pack * c)
    out2 = _final(parts2, y2, dis2, b1p, w2p, b2p)
    return out2.reshape(n, c)
